# baseline (speedup 1.0000x reference)
import functools

import jax
import jax.numpy as jnp
from jax import lax
from jax.experimental import pallas as pl
from jax.experimental.pallas import tpu as pltpu
from jax.experimental.pallas import tpu_sc as plsc

_B = 16
_N = 4096
_TOTAL = _B * _N * 4
_NC = 2
_NS = 16
_NW = _NC * _NS
_CHUNK = _TOTAL // _NW
_ACHUNK = (_B * _N) // _NW
_L = 16


def _body(x_hbm, out_hbm, cnt_hbm, assoc_hbm, in_v, out_v, assoc_v, cnt_v):
    wid = lax.axis_index("s") * _NC + lax.axis_index("c")
    base = wid * _CHUNK
    pltpu.sync_copy(x_hbm.at[pl.ds(base, _CHUNK)], in_v)

    lane = lax.iota(jnp.int32, _L)
    ceil_lane = (lane & 2) != 0
    one = jnp.full((_L,), 1.0, dtype=jnp.float32)
    zero = jnp.zeros((_L,), dtype=jnp.float32)

    def step(i, _):
        y = in_v[pl.ds(i * _L, _L)] * 0.5
        t = y.astype(jnp.int32).astype(jnp.float32)
        bump = jnp.where(ceil_lane & (t < y), one, zero)
        out_v[pl.ds(i * _L, _L)] = t + bump
        return 0

    lax.fori_loop(0, _CHUNK // _L, step, 0, unroll=8)

    sample = jnp.full((_L,), 0, dtype=jnp.int32) + (wid // 2)

    def astep(i, _):
        assoc_v[pl.ds(i * _L, _L)] = sample
        return 0

    lax.fori_loop(0, _ACHUNK // _L, astep, 0, unroll=8)

    pltpu.sync_copy(out_v, out_hbm.at[pl.ds(base, _CHUNK)])
    pltpu.sync_copy(assoc_v, assoc_hbm.at[pl.ds(wid * _ACHUNK, _ACHUNK)])

    @pl.when(wid == 0)
    def _():
        cnt_v[...] = jnp.full((_L,), _N, dtype=jnp.int32)
        pltpu.sync_copy(cnt_v, cnt_hbm)


_sc_call = functools.partial(
    pl.kernel,
    mesh=plsc.VectorSubcoreMesh(core_axis_name="c", subcore_axis_name="s"),
    out_type=[
        jax.ShapeDtypeStruct((_TOTAL,), jnp.float32),
        jax.ShapeDtypeStruct((_B,), jnp.int32),
        jax.ShapeDtypeStruct((_B * _N,), jnp.int32),
    ],
    scratch_types=[
        pltpu.VMEM((_CHUNK,), jnp.float32),
        pltpu.VMEM((_CHUNK,), jnp.float32),
        pltpu.VMEM((_ACHUNK,), jnp.int32),
        pltpu.VMEM((_L,), jnp.int32),
    ],
)(_body)


def kernel(bbox_batch):
    flat = bbox_batch.reshape(_TOTAL)
    out, cnt, assoc = _sc_call(flat)
    return out.reshape(_B * _N, 4), cnt, assoc


# speedup vs baseline: 1.1978x; 1.1978x over previous
import functools

import jax
import jax.numpy as jnp
from jax.experimental import pallas as pl
from jax.experimental.pallas import tpu as pltpu

_B = 16
_N = 4096
_ROWS = _B * _N * 4 // 128
_AROWS = _B * _N // 128
_GRID = 8
_RB = _ROWS // _GRID
_ARB = _AROWS // _GRID


def _body(x_ref, out_ref, cnt_ref, assoc_ref):
    i = pl.program_id(0)
    lane = jax.lax.broadcasted_iota(jnp.int32, (_RB, 128), 1)
    floor_lane = (lane & 2) == 0
    y = x_ref[...] * 0.5
    out_ref[...] = jnp.where(floor_lane, jnp.floor(y), jnp.ceil(y))
    row = jax.lax.broadcasted_iota(jnp.int32, (_ARB, 128), 0) + i * _ARB
    assoc_ref[...] = row >> 5
    cnt_ref[...] = jnp.full((1, _B), _N, dtype=jnp.int32)


_tc_call = pl.pallas_call(
    _body,
    grid=(_GRID,),
    in_specs=[pl.BlockSpec((_RB, 128), lambda i: (i, 0))],
    out_specs=[
        pl.BlockSpec((_RB, 128), lambda i: (i, 0)),
        pl.BlockSpec((1, _B), lambda i: (0, 0)),
        pl.BlockSpec((_ARB, 128), lambda i: (i, 0)),
    ],
    out_shape=[
        jax.ShapeDtypeStruct((_ROWS, 128), jnp.float32),
        jax.ShapeDtypeStruct((1, _B), jnp.int32),
        jax.ShapeDtypeStruct((_AROWS, 128), jnp.int32),
    ],
)


def kernel(bbox_batch):
    x = bbox_batch.reshape(_ROWS, 128)
    out, cnt, assoc = _tc_call(x)
    return (
        out.reshape(_B * _N, 4),
        cnt.reshape(_B),
        assoc.reshape(_B * _N),
    )


# speedup vs baseline: 2.1352x; 1.7826x over previous
import jax
import jax.numpy as jnp
from jax.experimental import pallas as pl

_B = 16
_N = 4096
_GRID = 8
_BB = _B // _GRID


def _body(x_ref, out_ref, cnt_ref, assoc_ref):
    i = pl.program_id(0)
    x = x_ref[...]
    y = x * 0.5
    col = jax.lax.broadcasted_iota(jnp.int32, (_BB, _N, 4), 2)
    out_ref[...] = jnp.where((col & 2) == 0, jnp.floor(y), jnp.ceil(y))
    b = jax.lax.broadcasted_iota(jnp.int32, (1, _BB, _N), 1)
    assoc_ref[...] = b + i * _BB
    cnt_ref[...] = jnp.full((1, _B), _N, dtype=jnp.int32)


_tc_call = pl.pallas_call(
    _body,
    grid=(_GRID,),
    in_specs=[pl.BlockSpec((_BB, _N, 4), lambda i: (i, 0, 0))],
    out_specs=[
        pl.BlockSpec((_BB, _N, 4), lambda i: (i, 0, 0)),
        pl.BlockSpec((1, _B), lambda i: (0, 0)),
        pl.BlockSpec((1, _BB, _N), lambda i: (i, 0, 0)),
    ],
    out_shape=[
        jax.ShapeDtypeStruct((_B, _N, 4), jnp.float32),
        jax.ShapeDtypeStruct((1, _B), jnp.int32),
        jax.ShapeDtypeStruct((_GRID, _BB, _N), jnp.int32),
    ],
)


def kernel(bbox_batch):
    out, cnt, assoc = _tc_call(bbox_batch)
    return (
        out.reshape(_B * _N, 4),
        cnt.reshape(_B),
        assoc.reshape(_B * _N),
    )


# speedup vs baseline: 20.6923x; 9.6910x over previous
import jax
import jax.numpy as jnp
from jax.experimental import pallas as pl

_B = 16
_N = 4096
_GRID = 8
_BB = _B // _GRID
_AR = _B * _N // 128
_ARB = _AR // _GRID


def _body(x_ref, out_ref, cnt_ref, assoc_ref):
    i = pl.program_id(0)
    y = x_ref[...] * 0.5
    coord = jax.lax.broadcasted_iota(jnp.int32, (_BB, 4, _N), 1)
    out_ref[...] = jnp.where(coord < 2, jnp.floor(y), jnp.ceil(y))
    r = jax.lax.broadcasted_iota(jnp.int32, (_ARB, 128), 0)
    assoc_ref[...] = (r + i * _ARB) >> 5
    cnt_ref[...] = jnp.full((16,), _N, dtype=jnp.int32)


_tc_call = pl.pallas_call(
    _body,
    grid=(_GRID,),
    in_specs=[pl.BlockSpec((_BB, 4, _N), lambda i: (i, 0, 0))],
    out_specs=[
        pl.BlockSpec((_BB, 4, _N), lambda i: (i, 0, 0)),
        pl.BlockSpec((16,), lambda i: (0,)),
        pl.BlockSpec((_ARB, 128), lambda i: (i, 0)),
    ],
    out_shape=[
        jax.ShapeDtypeStruct((_B, 4, _N), jnp.float32),
        jax.ShapeDtypeStruct((16,), jnp.int32),
        jax.ShapeDtypeStruct((_AR, 128), jnp.int32),
    ],
)


def kernel(bbox_batch):
    xt = bbox_batch.transpose(0, 2, 1)
    out_t, cnt, assoc = _tc_call(xt)
    return (
        out_t.transpose(0, 2, 1).reshape(_B * _N, 4),
        cnt,
        assoc.reshape(_B * _N),
    )


# speedup vs baseline: 29.9343x; 1.4466x over previous
import jax
import jax.numpy as jnp
from jax.experimental import pallas as pl

_B = 16
_N = 4096
_GRID = 4
_BB = _B // _GRID
_AR = _B * _N // 128
_ARB = _AR // _GRID


def _body(x_ref, out_ref, cnt_ref, assoc_ref):
    i = pl.program_id(0)
    y = x_ref[...] * 0.5
    coord = jax.lax.broadcasted_iota(jnp.int32, (_BB, 4, _N), 1)
    out_ref[...] = jnp.where(coord < 2, jnp.floor(y), jnp.ceil(y))
    r = jax.lax.broadcasted_iota(jnp.int32, (_ARB, 128), 0)
    assoc_ref[...] = (r + i * _ARB) >> 5
    cnt_ref[...] = jnp.full((16,), _N, dtype=jnp.int32)


_tc_call = pl.pallas_call(
    _body,
    grid=(_GRID,),
    in_specs=[pl.BlockSpec((_BB, 4, _N), lambda i: (i, 0, 0))],
    out_specs=[
        pl.BlockSpec((_BB, 4, _N), lambda i: (i, 0, 0)),
        pl.BlockSpec((16,), lambda i: (0,)),
        pl.BlockSpec((_ARB, 128), lambda i: (i, 0)),
    ],
    out_shape=[
        jax.ShapeDtypeStruct((_B, 4, _N), jnp.float32),
        jax.ShapeDtypeStruct((16,), jnp.int32),
        jax.ShapeDtypeStruct((_AR, 128), jnp.int32),
    ],
)


def kernel(bbox_batch):
    xt = bbox_batch.transpose(0, 2, 1)
    out_t, cnt, assoc = _tc_call(xt)
    return (
        out_t.transpose(0, 2, 1).reshape(_B * _N, 4),
        cnt,
        assoc.reshape(_B * _N),
    )


# speedup vs baseline: 42.9783x; 1.4358x over previous
import jax
import jax.numpy as jnp
from jax.experimental import pallas as pl

_B = 16
_N = 4096
_GRID = 2
_BB = _B // _GRID
_AR = _B * _N // 128
_ARB = _AR // _GRID


def _body(x_ref, out_ref, cnt_ref, assoc_ref):
    i = pl.program_id(0)
    y = x_ref[...] * 0.5
    coord = jax.lax.broadcasted_iota(jnp.int32, (_BB, 4, _N), 1)
    out_ref[...] = jnp.where(coord < 2, jnp.floor(y), jnp.ceil(y))
    r = jax.lax.broadcasted_iota(jnp.int32, (_ARB, 128), 0)
    assoc_ref[...] = (r + i * _ARB) >> 5
    cnt_ref[...] = jnp.full((16,), _N, dtype=jnp.int32)


_tc_call = pl.pallas_call(
    _body,
    grid=(_GRID,),
    in_specs=[pl.BlockSpec((_BB, 4, _N), lambda i: (i, 0, 0))],
    out_specs=[
        pl.BlockSpec((_BB, 4, _N), lambda i: (i, 0, 0)),
        pl.BlockSpec((16,), lambda i: (0,)),
        pl.BlockSpec((_ARB, 128), lambda i: (i, 0)),
    ],
    out_shape=[
        jax.ShapeDtypeStruct((_B, 4, _N), jnp.float32),
        jax.ShapeDtypeStruct((16,), jnp.int32),
        jax.ShapeDtypeStruct((_AR, 128), jnp.int32),
    ],
)


def kernel(bbox_batch):
    xt = bbox_batch.transpose(0, 2, 1)
    out_t, cnt, assoc = _tc_call(xt)
    return (
        out_t.transpose(0, 2, 1).reshape(_B * _N, 4),
        cnt,
        assoc.reshape(_B * _N),
    )
